# baseline (speedup 1.0000x reference)
import functools
import jax
import jax.numpy as jnp
from jax import lax
from jax.experimental import pallas as pl
from jax.experimental.pallas import tpu as pltpu
from jax.experimental.pallas import tpu_sc as plsc

NC, NS, L = 2, 16, 16
NW = NC * NS
EMBED_DIM = 64
SCALE = 8.0

BATCH = 4096 * 200
B_PER_W = BATCH // NW
GATHER_W = 128
N_CHUNKS = B_PER_W // GATHER_W


def _make_kernel():
    mesh = plsc.VectorSubcoreMesh(
        core_axis_name="c", subcore_axis_name="s", num_cores=NC, num_subcores=NS
    )

    @functools.partial(
        pl.kernel,
        out_type=jax.ShapeDtypeStruct((NW, N_CHUNKS, GATHER_W, EMBED_DIM), jnp.float32),
        mesh=mesh,
        scratch_types=[
            pltpu.VMEM((N_CHUNKS, GATHER_W), jnp.int32),
            pltpu.VMEM((GATHER_W, EMBED_DIM), jnp.float32),
            pltpu.SemaphoreType.DMA,
        ],
        compiler_params=pltpu.CompilerParams(use_tc_tiling_on_sc=False),
    )
    def emb_kernel(tokens_hbm, table_hbm, out_hbm, idx_v, rows_v, sem):
        wid = lax.axis_index("s") * NC + lax.axis_index("c")
        pltpu.sync_copy(tokens_hbm.at[wid], idx_v)

        def chunk(j, _):
            pltpu.async_copy(table_hbm.at[idx_v.at[j]], rows_v, sem).wait()

            def scale_row(i, _):
                for t in range(EMBED_DIM // L):
                    sl = pl.ds(t * L, L)
                    rows_v[i, sl] = rows_v[i, sl] * SCALE
                return 0

            lax.fori_loop(0, GATHER_W, scale_row, 0)
            pltpu.sync_copy(rows_v, out_hbm.at[wid, j])
            return 0

        lax.fori_loop(0, N_CHUNKS, chunk, 0)

    return emb_kernel


_emb_kernel = _make_kernel()


@jax.jit
def kernel(tokens, table):
    idx = tokens.reshape(NW, N_CHUNKS, GATHER_W).astype(jnp.int32)
    out = _emb_kernel(idx, table)
    return out.reshape(4096, 200, EMBED_DIM)


# speedup vs baseline: 1.2060x; 1.2060x over previous
import functools
import jax
import jax.numpy as jnp
from jax import lax
from jax.experimental import pallas as pl
from jax.experimental.pallas import tpu as pltpu
from jax.experimental.pallas import tpu_sc as plsc

NC, NS, L = 2, 16, 16
NW = NC * NS
EMBED_DIM = 64
SCALE = 8.0

BATCH = 4096 * 200
B_PER_W = BATCH // NW
GATHER_W = 128
CHUNK = 512
G_PER_CHUNK = CHUNK // GATHER_W
N_CHUNKS = B_PER_W // CHUNK
N_IDX_ROWS = B_PER_W // GATHER_W
NBUF = 2


def _make_kernel():
    mesh = plsc.VectorSubcoreMesh(
        core_axis_name="c", subcore_axis_name="s", num_cores=NC, num_subcores=NS
    )

    @functools.partial(
        pl.kernel,
        out_type=jax.ShapeDtypeStruct((NW, N_CHUNKS, CHUNK, EMBED_DIM), jnp.float32),
        mesh=mesh,
        scratch_types=[
            pltpu.VMEM((N_IDX_ROWS, GATHER_W), jnp.int32),
            pltpu.VMEM((NBUF, CHUNK, EMBED_DIM), jnp.float32),
            pltpu.SemaphoreType.DMA,
            pltpu.SemaphoreType.DMA,
            pltpu.SemaphoreType.DMA,
            pltpu.SemaphoreType.DMA,
        ],
        compiler_params=pltpu.CompilerParams(use_tc_tiling_on_sc=False),
    )
    def emb_kernel(tokens_hbm, table_hbm, out_hbm, idx_v, rows_v, g0, g1, s0, s1):
        wid = lax.axis_index("s") * NC + lax.axis_index("c")
        gsem = (g0, g1)
        ssem = (s0, s1)
        pltpu.sync_copy(tokens_hbm.at[wid], idx_v)

        def fire_gathers(b, g):
            for k in range(G_PER_CHUNK):
                pltpu.async_copy(
                    table_hbm.at[idx_v.at[g * G_PER_CHUNK + k]],
                    rows_v.at[b, pl.ds(k * GATHER_W, GATHER_W)],
                    gsem[b],
                )

        def drain_gathers(b):
            for k in range(G_PER_CHUNK):
                pltpu.make_async_copy(
                    table_hbm.at[idx_v.at[0]],
                    rows_v.at[b, pl.ds(k * GATHER_W, GATHER_W)],
                    gsem[b],
                ).wait()

        def scale_buf(b):
            @pl.loop(0, CHUNK, unroll=4)
            def _(i):
                for t in range(EMBED_DIM // L):
                    sl = pl.ds(t * L, L)
                    rows_v[b, i, sl] = rows_v[b, i, sl] * SCALE

        for b in range(NBUF):
            fire_gathers(b, b)

        @pl.loop(0, N_CHUNKS // NBUF)
        def _(o):
            for b in range(NBUF):
                g = o * NBUF + b
                drain_gathers(b)
                scale_buf(b)
                pltpu.async_copy(rows_v.at[b], out_hbm.at[wid, g], ssem[b])
                pltpu.make_async_copy(rows_v.at[b], out_hbm.at[wid, 0], ssem[b]).wait()

                @pl.when(g + NBUF < N_CHUNKS)
                def _():
                    fire_gathers(b, g + NBUF)

    return emb_kernel


_emb_kernel = _make_kernel()


@jax.jit
def kernel(tokens, table):
    idx = tokens.reshape(NW, N_IDX_ROWS, GATHER_W).astype(jnp.int32)
    out = _emb_kernel(idx, table)
    return out.reshape(4096, 200, EMBED_DIM)
